# baseline (speedup 1.0000x reference)
import functools

import numpy as np

import jax
import jax.numpy as jnp
from jax import lax
from jax.experimental import pallas as pl
from jax.experimental.pallas import tpu as pltpu
from jax.experimental.pallas import tpu_sc as plsc

B = 512
C = 64
NLOUT = 16
L = 16
P3 = 4
P2 = 2
P1 = 1
QUAD = 32
NQ = B // QUAD
W = QUAD * C
NPAIR = (L * (L + 1)) // 2

_JJ, _KK = np.triu_indices(L)


def _tc_body(xq_ref, w3_ref, w2_ref, w1_ref, u3_ref, u2_ref, u1_ref, out_ref):
    def one(q):
        xq = xq_ref[q]
        w3 = w3_ref[q]
        w2 = w2_ref[q]
        w1 = w1_ref[q]

        t = u1_ref[:, :] * w1

        xx = jnp.concatenate(
            [xq * xq[j:j + 1, :] for j in range(L)], axis=0
        ).astype(jnp.bfloat16)
        for p in range(P3):
            t = t + jnp.dot(u3_ref[p], xx,
                            preferred_element_type=jnp.float32) * w3[p:p + 1, :]

        zw2 = jnp.concatenate([xq * w2[p:p + 1, :] for p in range(P2)], axis=0)
        t = t + jnp.dot(u2_ref[:, :], zw2, preferred_element_type=jnp.float32)

        acc = t[0:L, :] * xq[0:1, :]
        for i in range(1, L):
            acc = acc + t[i * L:(i + 1) * L, :] * xq[i:i + 1, :]
        out_ref[q] = acc

    def step(qq, carry):
        one(qq * 2)
        one(qq * 2 + 1)
        return carry

    lax.fori_loop(0, NQ // 2, step, 0)


def _sc_weight_gather(table, idx):
    V, D = table.shape
    nb = B // 32
    mesh = plsc.VectorSubcoreMesh(core_axis_name="c", subcore_axis_name="s")

    @functools.partial(
        pl.kernel, mesh=mesh,
        out_type=jax.ShapeDtypeStruct((B, D), jnp.float32),
        scratch_types=[
            pltpu.VMEM((nb,), jnp.int32),
            pltpu.VMEM((nb, D), jnp.float32),
            pltpu.SemaphoreType.DMA,
        ],
    )
    def gather_k(table_hbm, idx_hbm, out_hbm, idx_v, rows_v, sem):
        wid = lax.axis_index("s") * 2 + lax.axis_index("c")
        base = wid * nb
        pltpu.sync_copy(idx_hbm.at[pl.ds(base, nb)], idx_v)
        pltpu.async_copy(table_hbm.at[idx_v], rows_v, sem).wait()
        pltpu.sync_copy(rows_v, out_hbm.at[pl.ds(base, nb)])

    return gather_k(table, idx)


def kernel(x, atom_types, U3, U2, U1, W3, W2, W1):
    table = jnp.concatenate(
        [W3.reshape(W3.shape[0], P3 * C),
         W2.reshape(W2.shape[0], P2 * C),
         W1.reshape(W1.shape[0], P1 * C),
         jnp.zeros((W1.shape[0], C), jnp.float32)], axis=1)
    gathered = _sc_weight_gather(table, atom_types)
    W3g = gathered[:, :P3 * C].reshape(B, P3, C)
    W2g = gathered[:, P3 * C:(P3 + P2) * C].reshape(B, P2, C)
    W1g = gathered[:, (P3 + P2) * C:(P3 + P2 + P1) * C].reshape(B, P1, C)

    def quad_cols(a):
        n = a.shape[1]
        return a.reshape(NQ, QUAD, n, C).transpose(0, 2, 1, 3).reshape(NQ, n, W)

    xq = quad_cols(x)
    w3q = quad_cols(W3g)
    w2q = quad_cols(W2g)
    w1q = quad_cols(W1g)

    u3t = (U3.transpose(4, 1, 0, 2, 3)
           .reshape(P3, NLOUT * L, L * L).astype(jnp.bfloat16))

    u2r = U2.transpose(1, 0, 3, 2).reshape(NLOUT * L, P2 * L)
    u1v = U1.transpose(1, 0, 2).reshape(NLOUT * L, P1)

    vm = pl.BlockSpec(memory_space=pltpu.VMEM)
    out = pl.pallas_call(
        _tc_body,
        in_specs=[vm] * 7,
        out_specs=vm,
        out_shape=jax.ShapeDtypeStruct((NQ, NLOUT, W), jnp.float32),
    )(xq, w3q, w2q, w1q, u3t, u2r, u1v)

    return out.reshape(NQ, NLOUT, QUAD, C).transpose(0, 2, 1, 3).reshape(B, NLOUT, C)


# speedup vs baseline: 1.0285x; 1.0285x over previous
import functools

import numpy as np

import jax
import jax.numpy as jnp
from jax import lax
from jax.experimental import pallas as pl
from jax.experimental.pallas import tpu as pltpu
from jax.experimental.pallas import tpu_sc as plsc

B = 512
C = 64
NLOUT = 16
L = 16
P3 = 4
P2 = 2
P1 = 1
QUAD = 8
NQ = B // QUAD
W = QUAD * C
NPAIR = (L * (L + 1)) // 2

_JJ, _KK = np.triu_indices(L)


def _tc_body(xq_ref, w3_ref, w2_ref, w1_ref, u3_ref, u2_ref, u1_ref, out_ref):
    def one(q):
        xq = xq_ref[q]
        w3 = w3_ref[q]
        w2 = w2_ref[q]
        w1 = w1_ref[q]

        t = u1_ref[:, :] * w1

        xx = jnp.concatenate(
            [xq * xq[j:j + 1, :] for j in range(L)], axis=0
        ).astype(jnp.bfloat16)
        for p in range(P3):
            t = t + jnp.dot(u3_ref[p], xx,
                            preferred_element_type=jnp.float32) * w3[p:p + 1, :]

        zw2 = jnp.concatenate([xq * w2[p:p + 1, :] for p in range(P2)], axis=0)
        t = t + jnp.dot(u2_ref[:, :], zw2, preferred_element_type=jnp.float32)

        acc = t[0:L, :] * xq[0:1, :]
        for i in range(1, L):
            acc = acc + t[i * L:(i + 1) * L, :] * xq[i:i + 1, :]
        out_ref[q] = acc

    def step(qq, carry):
        one(qq * 4)
        one(qq * 4 + 1)
        one(qq * 4 + 2)
        one(qq * 4 + 3)
        return carry

    lax.fori_loop(0, NQ // 4, step, 0)


def _sc_weight_gather(table, idx):
    V, D = table.shape
    nb = B // 32
    mesh = plsc.VectorSubcoreMesh(core_axis_name="c", subcore_axis_name="s")

    @functools.partial(
        pl.kernel, mesh=mesh,
        out_type=jax.ShapeDtypeStruct((B, D), jnp.float32),
        scratch_types=[
            pltpu.VMEM((nb,), jnp.int32),
            pltpu.VMEM((nb, D), jnp.float32),
            pltpu.SemaphoreType.DMA,
        ],
    )
    def gather_k(table_hbm, idx_hbm, out_hbm, idx_v, rows_v, sem):
        wid = lax.axis_index("s") * 2 + lax.axis_index("c")
        base = wid * nb
        pltpu.sync_copy(idx_hbm.at[pl.ds(base, nb)], idx_v)
        pltpu.async_copy(table_hbm.at[idx_v], rows_v, sem).wait()
        pltpu.sync_copy(rows_v, out_hbm.at[pl.ds(base, nb)])

    return gather_k(table, idx)


def kernel(x, atom_types, U3, U2, U1, W3, W2, W1):
    table = jnp.concatenate(
        [W3.reshape(W3.shape[0], P3 * C),
         W2.reshape(W2.shape[0], P2 * C),
         W1.reshape(W1.shape[0], P1 * C),
         jnp.zeros((W1.shape[0], C), jnp.float32)], axis=1)
    gathered = _sc_weight_gather(table, atom_types)
    W3g = gathered[:, :P3 * C].reshape(B, P3, C)
    W2g = gathered[:, P3 * C:(P3 + P2) * C].reshape(B, P2, C)
    W1g = gathered[:, (P3 + P2) * C:(P3 + P2 + P1) * C].reshape(B, P1, C)

    def quad_cols(a):
        n = a.shape[1]
        return a.reshape(NQ, QUAD, n, C).transpose(0, 2, 1, 3).reshape(NQ, n, W)

    xq = quad_cols(x)
    w3q = quad_cols(W3g)
    w2q = quad_cols(W2g)
    w1q = quad_cols(W1g)

    u3t = (U3.transpose(4, 1, 0, 2, 3)
           .reshape(P3, NLOUT * L, L * L).astype(jnp.bfloat16))

    u2r = U2.transpose(1, 0, 3, 2).reshape(NLOUT * L, P2 * L)
    u1v = U1.transpose(1, 0, 2).reshape(NLOUT * L, P1)

    vm = pl.BlockSpec(memory_space=pltpu.VMEM)
    out = pl.pallas_call(
        _tc_body,
        in_specs=[vm] * 7,
        out_specs=vm,
        out_shape=jax.ShapeDtypeStruct((NQ, NLOUT, W), jnp.float32),
    )(xq, w3q, w2q, w1q, u3t, u2r, u1v)

    return out.reshape(NQ, NLOUT, QUAD, C).transpose(0, 2, 1, 3).reshape(B, NLOUT, C)


# speedup vs baseline: 1.0438x; 1.0148x over previous
import functools

import numpy as np

import jax
import jax.numpy as jnp
from jax import lax
from jax.experimental import pallas as pl
from jax.experimental.pallas import tpu as pltpu
from jax.experimental.pallas import tpu_sc as plsc

B = 512
C = 64
NLOUT = 16
L = 16
P3 = 4
P2 = 2
P1 = 1
QUAD = 8
NQ = B // QUAD
W = QUAD * C
NPAIR = (L * (L + 1)) // 2

_JJ, _KK = np.triu_indices(L)


def _tc_body(xq_ref, w3_ref, w2_ref, w1_ref, u3_ref, u2_ref, u1_ref, out_ref):
    def one(q):
        xq = xq_ref[q]
        w3 = w3_ref[q]
        w2 = w2_ref[q]
        w1 = w1_ref[q]

        t = u1_ref[:, :] * w1

        xx = jnp.concatenate(
            [xq * xq[j:j + 1, :] for j in range(L)], axis=0
        ).astype(jnp.bfloat16)
        for p in range(P3):
            t = t + jnp.dot(u3_ref[p], xx,
                            preferred_element_type=jnp.float32) * w3[p:p + 1, :]

        zw2 = jnp.concatenate([xq * w2[p:p + 1, :] for p in range(P2)], axis=0)
        t = t + jnp.dot(u2_ref[:, :], zw2, preferred_element_type=jnp.float32)

        acc = t[0:L, :] * xq[0:1, :]
        for i in range(1, L):
            acc = acc + t[i * L:(i + 1) * L, :] * xq[i:i + 1, :]
        out_ref[q] = acc

    def step(qq, carry):
        for h in range(8):
            one(qq * 8 + h)
        return carry

    lax.fori_loop(0, NQ // 8, step, 0)


def _sc_weight_gather(table, idx):
    V, D = table.shape
    nb = B // 32
    mesh = plsc.VectorSubcoreMesh(core_axis_name="c", subcore_axis_name="s")

    @functools.partial(
        pl.kernel, mesh=mesh,
        out_type=jax.ShapeDtypeStruct((B, D), jnp.float32),
        scratch_types=[
            pltpu.VMEM((nb,), jnp.int32),
            pltpu.VMEM((nb, D), jnp.float32),
            pltpu.SemaphoreType.DMA,
        ],
    )
    def gather_k(table_hbm, idx_hbm, out_hbm, idx_v, rows_v, sem):
        wid = lax.axis_index("s") * 2 + lax.axis_index("c")
        base = wid * nb
        pltpu.sync_copy(idx_hbm.at[pl.ds(base, nb)], idx_v)
        pltpu.async_copy(table_hbm.at[idx_v], rows_v, sem).wait()
        pltpu.sync_copy(rows_v, out_hbm.at[pl.ds(base, nb)])

    return gather_k(table, idx)


def kernel(x, atom_types, U3, U2, U1, W3, W2, W1):
    table = jnp.concatenate(
        [W3.reshape(W3.shape[0], P3 * C),
         W2.reshape(W2.shape[0], P2 * C),
         W1.reshape(W1.shape[0], P1 * C),
         jnp.zeros((W1.shape[0], C), jnp.float32)], axis=1)
    gathered = _sc_weight_gather(table, atom_types)
    W3g = gathered[:, :P3 * C].reshape(B, P3, C)
    W2g = gathered[:, P3 * C:(P3 + P2) * C].reshape(B, P2, C)
    W1g = gathered[:, (P3 + P2) * C:(P3 + P2 + P1) * C].reshape(B, P1, C)

    def quad_cols(a):
        n = a.shape[1]
        return a.reshape(NQ, QUAD, n, C).transpose(0, 2, 1, 3).reshape(NQ, n, W)

    xq = quad_cols(x)
    w3q = quad_cols(W3g)
    w2q = quad_cols(W2g)
    w1q = quad_cols(W1g)

    u3t = (U3.transpose(4, 1, 0, 2, 3)
           .reshape(P3, NLOUT * L, L * L).astype(jnp.bfloat16))

    u2r = U2.transpose(1, 0, 3, 2).reshape(NLOUT * L, P2 * L)
    u1v = U1.transpose(1, 0, 2).reshape(NLOUT * L, P1)

    vm = pl.BlockSpec(memory_space=pltpu.VMEM)
    out = pl.pallas_call(
        _tc_body,
        in_specs=[vm] * 7,
        out_specs=vm,
        out_shape=jax.ShapeDtypeStruct((NQ, NLOUT, W), jnp.float32),
    )(xq, w3q, w2q, w1q, u3t, u2r, u1v)

    return out.reshape(NQ, NLOUT, QUAD, C).transpose(0, 2, 1, 3).reshape(B, NLOUT, C)
